# Initial kernel scaffold; baseline (speedup 1.0000x reference)
import jax
import jax.numpy as jnp
from jax.experimental import pallas as pl


def kernel(x, edge_index, edge_type, W1, q1, k1, g1, b1, W2, q2, k2, res_W, res_b, dec_W1, dec_b1, ln_g, ln_b, dec_W2, dec_b2, rel_diag):
    raise NotImplementedError("write your pallas kernel here")


# speedup vs baseline: 1.0027x; 1.0027x over previous
import functools

import jax
import jax.numpy as jnp
from jax.experimental import pallas as pl

N = 2048
E = 65536
D_IN = 256
R = 3
HID = 128
LAT = 64
HEADS = 4

ROW_BLK = 256


def _adj_pred_body(z_blk_ref, zt_ref, rd_ref, out_ref):
    z_blk = z_blk_ref[...]
    zt = zt_ref[...]
    for r in range(R):
        zr = z_blk * rd_ref[r][None, :]
        score = jnp.dot(zr, zt, preferred_element_type=jnp.float32)
        score = jnp.clip(score, -10.0, 10.0)
        p = jax.nn.sigmoid(score)
        out_ref[r, 0] = jnp.clip(p, 1e-6, 1.0 - 1e-6)


def _adj_preds(z, rel_diag):
    zt = z.T
    grid = (N // ROW_BLK,)
    return pl.pallas_call(
        _adj_pred_body,
        grid=grid,
        in_specs=[
            pl.BlockSpec((ROW_BLK, LAT), lambda i: (i, 0)),
            pl.BlockSpec((LAT, N), lambda i: (0, 0)),
            pl.BlockSpec((R, LAT), lambda i: (0, 0)),
        ],
        out_specs=pl.BlockSpec((R, 1, ROW_BLK, N), lambda i: (0, 0, i, 0)),
        out_shape=jax.ShapeDtypeStruct((R, 1, N, N), jnp.float32),
    )(z, zt, rel_diag)


def _segment_softmax_agg(logit, dst, h_src, n):
    m = jax.ops.segment_max(logit, dst, num_segments=n)
    m = jnp.where(jnp.isfinite(m), m, 0.0)
    e = jnp.exp(logit - m[dst])
    s = jax.ops.segment_sum(e, dst, num_segments=n)
    alpha = e / (s[dst] + 1e-16)
    return jax.ops.segment_sum(alpha[..., None] * h_src, dst, num_segments=n)


def _rgat(x, src, dst, et, W, aq, ak, heads, out_ch):
    xr = jnp.einsum('ni,rio->rno', x, W)
    h_src = xr[et, src].reshape(-1, heads, out_ch)
    h_dst = xr[et, dst].reshape(-1, heads, out_ch)
    logit = jax.nn.leaky_relu(
        (h_dst * aq[None]).sum(-1) + (h_src * ak[None]).sum(-1),
        negative_slope=0.2)
    out = _segment_softmax_agg(logit, dst, h_src, x.shape[0])
    return out.reshape(x.shape[0], heads * out_ch)


def kernel(x, edge_index, edge_type, W1, q1, k1, g1, b1, W2, q2, k2,
           res_W, res_b, dec_W1, dec_b1, ln_g, ln_b, dec_W2, dec_b2,
           rel_diag):
    src = edge_index[0]
    dst = edge_index[1]

    h = _rgat(x, src, dst, edge_type, W1, q1, k1, HEADS, HID // HEADS)
    mu = h.mean(0)
    var = h.var(0)
    h = (h - mu) / jnp.sqrt(var + 1e-5) * g1 + b1
    h = jax.nn.leaky_relu(h, negative_slope=0.2)

    z = _rgat(h, src, dst, edge_type, W2, q2, k2, 1, LAT)
    z = jnp.clip(z, -10.0, 10.0) + h @ res_W + res_b

    t = z @ dec_W1 + dec_b1
    t = (t - t.mean(-1, keepdims=True)) / jnp.sqrt(t.var(-1, keepdims=True) + 1e-5) * ln_g + ln_b
    t = jax.nn.relu(t)
    x_hat = t @ dec_W2 + dec_b2

    mask = jnp.ones((1, N), dtype=bool)
    adj_preds = _adj_preds(z, rel_diag)

    trues = []
    for r in range(R):
        w = (edge_type == r).astype(jnp.float32)
        adj_r = jnp.zeros((1, N, N), dtype=jnp.float32).at[0, src, dst].add(w)
        trues.append(adj_r)
    adj_true_rel = jnp.stack(trues, 0)

    graph_embedding = jnp.max(z, axis=0, keepdims=True)
    return (z, x_hat, adj_preds, adj_true_rel, mask, graph_embedding)


# speedup vs baseline: 1.0271x; 1.0243x over previous
import functools

import jax
import jax.numpy as jnp
from jax import lax
from jax.experimental import pallas as pl
from jax.experimental.pallas import tpu as pltpu
from jax.experimental.pallas import tpu_sc as plsc

N = 2048
E = 65536
D_IN = 256
R = 3
HID = 128
LAT = 64
HEADS = 4

ROW_BLK = 256


def _adj_pred_body(z_blk_ref, zt_ref, rd_ref, out_ref):
    z_blk = z_blk_ref[...]
    zt = zt_ref[...]
    for r in range(R):
        zr = z_blk * rd_ref[r][None, :]
        score = jnp.dot(zr, zt, preferred_element_type=jnp.float32)
        score = jnp.clip(score, -10.0, 10.0)
        p = jax.nn.sigmoid(score)
        out_ref[r, 0] = jnp.clip(p, 1e-6, 1.0 - 1e-6)


def _adj_preds(z, rel_diag):
    zt = z.T
    grid = (N // ROW_BLK,)
    return pl.pallas_call(
        _adj_pred_body,
        grid=grid,
        in_specs=[
            pl.BlockSpec((ROW_BLK, LAT), lambda i: (i, 0)),
            pl.BlockSpec((LAT, N), lambda i: (0, 0)),
            pl.BlockSpec((R, LAT), lambda i: (0, 0)),
        ],
        out_specs=pl.BlockSpec((R, 1, ROW_BLK, N), lambda i: (0, 0, i, 0)),
        out_shape=jax.ShapeDtypeStruct((R, 1, N, N), jnp.float32),
    )(z, zt, rel_diag)


def _edge_addr_body(src_ref, dst_ref, et_ref, f_ref):
    f_ref[...] = (et_ref[...] * N + src_ref[...]) * N + dst_ref[...]


def _edge_addrs(src, dst, et):
    s2 = src.reshape(64, 1024)
    d2 = dst.reshape(64, 1024)
    t2 = et.reshape(64, 1024)
    f = pl.pallas_call(
        _edge_addr_body,
        out_shape=jax.ShapeDtypeStruct((64, 1024), jnp.int32),
    )(s2, d2, t2)
    return f.reshape(E)


_ADJ_PAGE = 65536
_ADJ_ROUNDS = (R * N * N) // (32 * _ADJ_PAGE)
_EBLK = 8192


def _adj_true_body(f_hbm, out_hbm, page, fbufs, sem0, sem1):
    wid = lax.axis_index("s") * 2 + lax.axis_index("c")
    ones = jnp.ones((16,), jnp.float32)
    zeros16 = jnp.zeros((16,), jnp.float32)

    def round_body(k, _):
        base = (k * 32 + wid) * _ADJ_PAGE

        def zero_chunk(i, _):
            for u in range(8):
                page[pl.ds(i * 128 + u * 16, 16)] = zeros16
            return _
        lax.fori_loop(0, _ADJ_PAGE // 128, zero_chunk, None)
        cp0 = pltpu.async_copy(f_hbm.at[pl.ds(0, _EBLK)], fbufs.at[0], sem0)
        for b in range(E // _EBLK):
            cur = b % 2
            if b % 2 == 0:
                cp0.wait()
            else:
                cp1.wait()
            if b + 1 < E // _EBLK:
                nxt = (b + 1) % 2
                if nxt == 0:
                    cp0 = pltpu.async_copy(
                        f_hbm.at[pl.ds((b + 1) * _EBLK, _EBLK)],
                        fbufs.at[0], sem0)
                else:
                    cp1 = pltpu.async_copy(
                        f_hbm.at[pl.ds((b + 1) * _EBLK, _EBLK)],
                        fbufs.at[1], sem1)

            def chunk(i, _):
                for u in range(8):
                    fv = fbufs[cur, pl.ds(i * 128 + u * 16, 16)]
                    local = fv - base
                    m = plsc.bitcast(local, jnp.uint32) < jnp.uint32(_ADJ_PAGE)
                    plsc.addupdate_scatter(page, [local], ones, mask=m)
                return _
            lax.fori_loop(0, _EBLK // 128, chunk, None)
        pltpu.sync_copy(page, out_hbm.at[pl.ds(base, _ADJ_PAGE)])
        return _
    lax.fori_loop(0, _ADJ_ROUNDS, round_body, None)


def _adj_true(f):
    mesh = plsc.VectorSubcoreMesh(core_axis_name="c", subcore_axis_name="s")
    run = pl.kernel(
        _adj_true_body,
        out_type=jax.ShapeDtypeStruct((R * N * N,), jnp.float32),
        mesh=mesh,
        compiler_params=pltpu.CompilerParams(needs_layout_passes=False),
        scratch_types=[
            pltpu.VMEM((_ADJ_PAGE,), jnp.float32),
            pltpu.VMEM((2, _EBLK), jnp.int32),
            pltpu.SemaphoreType.DMA,
            pltpu.SemaphoreType.DMA,
        ],
    )
    return run(f).reshape(R, 1, N, N)


def _segment_softmax_agg(logit, dst, h_src, n):
    m = jax.ops.segment_max(logit, dst, num_segments=n)
    m = jnp.where(jnp.isfinite(m), m, 0.0)
    e = jnp.exp(logit - m[dst])
    s = jax.ops.segment_sum(e, dst, num_segments=n)
    alpha = e / (s[dst] + 1e-16)
    return jax.ops.segment_sum(alpha[..., None] * h_src, dst, num_segments=n)


def _rgat(x, src, dst, et, W, aq, ak, heads, out_ch):
    xr = jnp.einsum('ni,rio->rno', x, W)
    h_src = xr[et, src].reshape(-1, heads, out_ch)
    h_dst = xr[et, dst].reshape(-1, heads, out_ch)
    logit = jax.nn.leaky_relu(
        (h_dst * aq[None]).sum(-1) + (h_src * ak[None]).sum(-1),
        negative_slope=0.2)
    out = _segment_softmax_agg(logit, dst, h_src, x.shape[0])
    return out.reshape(x.shape[0], heads * out_ch)


def kernel(x, edge_index, edge_type, W1, q1, k1, g1, b1, W2, q2, k2,
           res_W, res_b, dec_W1, dec_b1, ln_g, ln_b, dec_W2, dec_b2,
           rel_diag):
    src = edge_index[0]
    dst = edge_index[1]

    h = _rgat(x, src, dst, edge_type, W1, q1, k1, HEADS, HID // HEADS)
    mu = h.mean(0)
    var = h.var(0)
    h = (h - mu) / jnp.sqrt(var + 1e-5) * g1 + b1
    h = jax.nn.leaky_relu(h, negative_slope=0.2)

    z = _rgat(h, src, dst, edge_type, W2, q2, k2, 1, LAT)
    z = jnp.clip(z, -10.0, 10.0) + h @ res_W + res_b

    t = z @ dec_W1 + dec_b1
    t = (t - t.mean(-1, keepdims=True)) / jnp.sqrt(t.var(-1, keepdims=True) + 1e-5) * ln_g + ln_b
    t = jax.nn.relu(t)
    x_hat = t @ dec_W2 + dec_b2

    mask = jnp.ones((1, N), dtype=bool)
    adj_preds = _adj_preds(z, rel_diag)

    f = _edge_addrs(src, dst, edge_type)
    adj_true_rel = _adj_true(f)

    graph_embedding = jnp.max(z, axis=0, keepdims=True)
    return (z, x_hat, adj_preds, adj_true_rel, mask, graph_embedding)


# speedup vs baseline: 25.8927x; 25.2089x over previous
import functools

import jax
import jax.numpy as jnp
from jax import lax
from jax.experimental import pallas as pl
from jax.experimental.pallas import tpu as pltpu
from jax.experimental.pallas import tpu_sc as plsc

N = 2048
E = 65536
D_IN = 256
R = 3
HID = 128
LAT = 64
HEADS = 4

ROW_BLK = 256


def _adj_pred_body(z_blk_ref, zt_ref, rd_ref, out_ref):
    z_blk = z_blk_ref[...]
    zt = zt_ref[...]
    for r in range(R):
        zr = z_blk * rd_ref[r][None, :]
        score = jnp.dot(zr, zt, preferred_element_type=jnp.float32)
        score = jnp.clip(score, -10.0, 10.0)
        p = jax.nn.sigmoid(score)
        out_ref[r, 0] = jnp.clip(p, 1e-6, 1.0 - 1e-6)


def _adj_preds(z, rel_diag):
    zt = z.T
    grid = (N // ROW_BLK,)
    return pl.pallas_call(
        _adj_pred_body,
        grid=grid,
        in_specs=[
            pl.BlockSpec((ROW_BLK, LAT), lambda i: (i, 0)),
            pl.BlockSpec((LAT, N), lambda i: (0, 0)),
            pl.BlockSpec((R, LAT), lambda i: (0, 0)),
        ],
        out_specs=pl.BlockSpec((R, 1, ROW_BLK, N), lambda i: (0, 0, i, 0)),
        out_shape=jax.ShapeDtypeStruct((R, 1, N, N), jnp.float32),
    )(z, zt, rel_diag)


def _edge_addr_body(src_ref, dst_ref, et_ref, f_ref, sidx_ref, didx_ref):
    base = et_ref[...] * N
    sidx_ref[...] = base + src_ref[...]
    didx_ref[...] = base + dst_ref[...]
    f_ref[...] = sidx_ref[...] * N + dst_ref[...]


def _edge_addrs(src, dst, et):
    s2 = src.reshape(64, 1024)
    d2 = dst.reshape(64, 1024)
    t2 = et.reshape(64, 1024)
    f, sidx, didx = pl.pallas_call(
        _edge_addr_body,
        out_shape=[jax.ShapeDtypeStruct((64, 1024), jnp.int32)] * 3,
    )(s2, d2, t2)
    return f.reshape(E), sidx.reshape(512, 128), didx.reshape(512, 128)


_ADJ_PAGE = 65536
_ADJ_ROUNDS = (R * N * N) // (32 * _ADJ_PAGE)
_EBLK = 8192


def _adj_true_body(f_hbm, out_hbm, page, fbufs, sem0, sem1):
    wid = lax.axis_index("s") * 2 + lax.axis_index("c")
    ones = jnp.ones((16,), jnp.float32)
    zeros16 = jnp.zeros((16,), jnp.float32)

    def round_body(k, _):
        base = (k * 32 + wid) * _ADJ_PAGE

        def zero_chunk(i, _):
            for u in range(8):
                page[pl.ds(i * 128 + u * 16, 16)] = zeros16
            return _
        lax.fori_loop(0, _ADJ_PAGE // 128, zero_chunk, None)
        cp0 = pltpu.async_copy(f_hbm.at[pl.ds(0, _EBLK)], fbufs.at[0], sem0)
        for b in range(E // _EBLK):
            cur = b % 2
            if b % 2 == 0:
                cp0.wait()
            else:
                cp1.wait()
            if b + 1 < E // _EBLK:
                nxt = (b + 1) % 2
                if nxt == 0:
                    cp0 = pltpu.async_copy(
                        f_hbm.at[pl.ds((b + 1) * _EBLK, _EBLK)],
                        fbufs.at[0], sem0)
                else:
                    cp1 = pltpu.async_copy(
                        f_hbm.at[pl.ds((b + 1) * _EBLK, _EBLK)],
                        fbufs.at[1], sem1)

            def chunk(i, _):
                for u in range(8):
                    fv = fbufs[cur, pl.ds(i * 128 + u * 16, 16)]
                    local = fv - base
                    m = plsc.bitcast(local, jnp.uint32) < jnp.uint32(_ADJ_PAGE)
                    plsc.addupdate_scatter(page, [local], ones, mask=m)
                return _
            lax.fori_loop(0, _EBLK // 128, chunk, None)
        pltpu.sync_copy(page, out_hbm.at[pl.ds(base, _ADJ_PAGE)])
        return _
    lax.fori_loop(0, _ADJ_ROUNDS, round_body, None)


def _adj_true(f):
    mesh = plsc.VectorSubcoreMesh(core_axis_name="c", subcore_axis_name="s")
    run = pl.kernel(
        _adj_true_body,
        out_type=jax.ShapeDtypeStruct((R * N * N,), jnp.float32),
        mesh=mesh,
        compiler_params=pltpu.CompilerParams(needs_layout_passes=False),
        scratch_types=[
            pltpu.VMEM((_ADJ_PAGE,), jnp.float32),
            pltpu.VMEM((2, _EBLK), jnp.int32),
            pltpu.SemaphoreType.DMA,
            pltpu.SemaphoreType.DMA,
        ],
    )
    return run(f).reshape(R, 1, N, N)


_EPT = E // 32
_NCH = _EPT // 128


def _edge_conv_body(C, H, sidx_hbm, didx_hbm, dst_hbm, qd_hbm, ks_hbm,
                    xr_hbm, num_out, den_out, qd_v, ks_v, den_v, sidx_v,
                    didx_v, dst_v, wbuf, rows, num_sh, sem):
    cid = lax.axis_index("c")
    sid = lax.axis_index("s")
    wid = sid * 2 + cid
    zeros16 = jnp.zeros((16,), jnp.float32)
    JPH = 32 // 16 if H > 1 else C // 16

    pltpu.sync_copy(qd_hbm, qd_v)
    pltpu.sync_copy(ks_hbm, ks_v)
    pltpu.sync_copy(sidx_hbm.at[pl.ds(wid * 16, 16)], sidx_v)
    pltpu.sync_copy(didx_hbm.at[pl.ds(wid * 16, 16)], didx_v)
    pltpu.sync_copy(dst_hbm.at[pl.ds(wid * 16, 16)], dst_v)

    def zden(i, _):
        den_v[pl.ds(i * 16, 16)] = zeros16
        return _
    lax.fori_loop(0, N * H // 16, zden, None)

    def zrow(r_, _):
        for u in range(C // 16):
            rows[r_, pl.ds(u * 16, 16)] = zeros16
        return _
    lax.fori_loop(0, 128, zrow, None)
    pltpu.sync_copy(rows, num_sh.at[pl.ds(sid * 128, 128)])
    plsc.subcore_barrier()

    def chunk(mc, _):
        pltpu.async_copy(xr_hbm.at[sidx_v.at[mc]], rows, sem).wait()
        for g in range(8):
            iv = sidx_v[mc, pl.ds(g * 16, 16)]
            jv = didx_v[mc, pl.ds(g * 16, 16)]
            dn = dst_v[mc, pl.ds(g * 16, 16)]
            for h in range(H):
                hf = jnp.full((16,), h, jnp.int32)
                q = plsc.load_gather(qd_v, [jv * H + hf])
                k = plsc.load_gather(ks_v, [iv * H + hf])
                l = q + k
                l = jnp.where(l >= 0.0, l, 0.2 * l)
                w = jnp.exp(l)
                wbuf[h, pl.ds(g * 16, 16)] = w
                plsc.addupdate_scatter(den_v, [dn * H + hf], w)
        def scale(e, _):
            ef = jnp.full((16,), e, jnp.int32)
            for h in range(H):
                hf = jnp.full((16,), h, jnp.int32)
                bc = plsc.load_gather(wbuf, [hf, ef])
                for u in range(JPH):
                    sl = pl.ds(h * (C // H) + u * 16, 16)
                    rows[e, sl] = rows[e, sl] * bc
            return _
        lax.fori_loop(0, 128, scale, None)
        pltpu.sync_copy(rows, num_sh.at[dst_v.at[mc]], add=True)
        return _
    lax.fori_loop(0, _NCH, chunk, None)

    plsc.subcore_barrier()
    pltpu.sync_copy(num_sh.at[pl.ds(sid * 128, 128)],
                    num_out.at[cid, pl.ds(sid * 128, 128)])
    pltpu.sync_copy(den_v, den_out.at[pl.ds(wid * N * H, N * H)])


def _edge_conv(C, H):
    mesh = plsc.VectorSubcoreMesh(core_axis_name="c", subcore_axis_name="s")
    body = functools.partial(_edge_conv_body, C, H)
    return pl.kernel(
        body,
        out_type=[
            jax.ShapeDtypeStruct((2, N, C), jnp.float32),
            jax.ShapeDtypeStruct((32 * N * H,), jnp.float32),
        ],
        mesh=mesh,
        compiler_params=pltpu.CompilerParams(needs_layout_passes=False),
        scratch_types=[
            pltpu.VMEM((R * N * H,), jnp.float32),
            pltpu.VMEM((R * N * H,), jnp.float32),
            pltpu.VMEM((N * H,), jnp.float32),
            pltpu.VMEM((16, 128), jnp.int32),
            pltpu.VMEM((16, 128), jnp.int32),
            pltpu.VMEM((16, 128), jnp.int32),
            pltpu.VMEM((H, 128), jnp.float32),
            pltpu.VMEM((128, C), jnp.float32),
            pltpu.VMEM_SHARED((N, C), jnp.float32),
            pltpu.SemaphoreType.DMA,
        ],
    )


def _rgat_sc(xrflat, q, k, sidx2, didx2, dst2, C, H):
    xr4 = xrflat.reshape(R * N, H, C // H)
    qd = jnp.einsum('nhc,hc->nh', xr4, q.reshape(H, C // H)).reshape(R * N * H)
    ks = jnp.einsum('nhc,hc->nh', xr4, k.reshape(H, C // H)).reshape(R * N * H)
    Cp = C
    if H == 1 and C < 128:
        Cp = 128
        xrflat = jnp.pad(xrflat, ((0, 0), (0, Cp - C)))
    num_parts, den_parts = _edge_conv(Cp, H)(
        sidx2, didx2, dst2, qd, ks, xrflat)
    num = num_parts[0] + num_parts[1]
    den = den_parts.reshape(32, N, H).sum(0)
    out = num.reshape(N, H, Cp // H) / (den[:, :, None] + 1e-16)
    return out.reshape(N, Cp)[:, :C]


def _segment_softmax_agg(logit, dst, h_src, n):
    m = jax.ops.segment_max(logit, dst, num_segments=n)
    m = jnp.where(jnp.isfinite(m), m, 0.0)
    e = jnp.exp(logit - m[dst])
    s = jax.ops.segment_sum(e, dst, num_segments=n)
    alpha = e / (s[dst] + 1e-16)
    return jax.ops.segment_sum(alpha[..., None] * h_src, dst, num_segments=n)


def _rgat(x, src, dst, et, W, aq, ak, heads, out_ch):
    xr = jnp.einsum('ni,rio->rno', x, W)
    h_src = xr[et, src].reshape(-1, heads, out_ch)
    h_dst = xr[et, dst].reshape(-1, heads, out_ch)
    logit = jax.nn.leaky_relu(
        (h_dst * aq[None]).sum(-1) + (h_src * ak[None]).sum(-1),
        negative_slope=0.2)
    out = _segment_softmax_agg(logit, dst, h_src, x.shape[0])
    return out.reshape(x.shape[0], heads * out_ch)


def kernel(x, edge_index, edge_type, W1, q1, k1, g1, b1, W2, q2, k2,
           res_W, res_b, dec_W1, dec_b1, ln_g, ln_b, dec_W2, dec_b2,
           rel_diag):
    src = edge_index[0]
    dst = edge_index[1]
    f, sidx2, didx2 = _edge_addrs(src, dst, edge_type)
    dst2 = dst.reshape(512, 128)

    xr1 = jnp.einsum('ni,rio->rno', x, W1).reshape(R * N, HID)
    hconv = _rgat_sc(xr1, q1, k1, sidx2, didx2, dst2, HID, HEADS)
    mu = hconv.mean(0)
    var = hconv.var(0)
    h = (hconv - mu) / jnp.sqrt(var + 1e-5) * g1 + b1
    h = jax.nn.leaky_relu(h, negative_slope=0.2)

    xr2 = jnp.einsum('ni,rio->rno', h, W2).reshape(R * N, LAT)
    zconv = _rgat_sc(xr2, q2, k2, sidx2, didx2, dst2, LAT, 1)
    z = jnp.clip(zconv, -10.0, 10.0) + h @ res_W + res_b

    t = z @ dec_W1 + dec_b1
    t = (t - t.mean(-1, keepdims=True)) / jnp.sqrt(t.var(-1, keepdims=True) + 1e-5) * ln_g + ln_b
    t = jax.nn.relu(t)
    x_hat = t @ dec_W2 + dec_b2

    mask = jnp.ones((1, N), dtype=bool)
    adj_preds = _adj_preds(z, rel_diag)

    adj_true_rel = _adj_true(f)

    graph_embedding = jnp.max(z, axis=0, keepdims=True)
    return (z, x_hat, adj_preds, adj_true_rel, mask, graph_embedding)
